# baseline (speedup 1.0000x reference)
import functools

import jax
import jax.numpy as jnp
from jax import lax
from jax.experimental import pallas as pl
from jax.experimental.pallas import tpu as pltpu
from jax.experimental.pallas import tpu_sc as plsc

N_NODES = 10000
N_EDGES = 320000
D = 128
H2 = 768
NC, NS = 2, 16
NW = NC * NS
EPW = N_EDGES // NW
CH = 80
NCHUNK = EPW // CH
N_PAD = 10112
RPT = N_PAD // NS
NB = 3
LA = NB - 1

PH = 25
NPH = NCHUNK // PH
CNT_LAG = 6


def _sc_aggregate_impl(x_hbm, src_hbm, dst_hbm, zrow_hbm, zcnt_hbm, ones_hbm,
                       sums_out, cnt_out,
                       srcb0, srcb1, dstb0, dstb1, rows0, rows1, rows2, ones_v,
                       semg0, semg1, semg2, sems0, sems1, sems2, semi, semc,
                       sh_sums, sh_cnt):
    cid = lax.axis_index("c")
    sid = lax.axis_index("s")
    wid = sid * NC + cid
    srcb = (srcb0, srcb1)
    dstb = (dstb0, dstb1)
    rows = (rows0, rows1, rows2)
    semg = (semg0, semg1, semg2)
    sems = (sems0, sems1, sems2)

    pltpu.sync_copy(zrow_hbm, rows0)
    pltpu.sync_copy(zcnt_hbm, ones_v)
    for z in range(8):
        w = CH if z < 7 else RPT - 7 * CH
        off = sid * RPT + z * CH
        pltpu.sync_copy(rows0.at[pl.ds(0, w)], sh_sums.at[pl.ds(off, w)])
        pltpu.sync_copy(ones_v.at[pl.ds(0, w)], sh_cnt.at[pl.ds(off, w)])
    pltpu.sync_copy(ones_hbm, ones_v)
    plsc.subcore_barrier()

    pltpu.sync_copy(src_hbm.at[wid, pl.ds(0, PH)], srcb0)
    pltpu.sync_copy(dst_hbm.at[wid, pl.ds(0, PH)], dstb0)
    desc_g = {}
    desc_s = {}
    desc_c = {}
    desc_i = None
    for k in range(LA):
        desc_g[k] = pltpu.async_copy(x_hbm.at[srcb0.at[k]],
                                     rows[k % NB], semg[k % NB])
    for c in range(NCHUNK):
        b = c % NB
        ph = c // PH
        pb = ph & 1
        j = c % PH
        if j == 0 and ph + 1 < NPH:
            for k in sorted(desc_s):
                desc_s.pop(k).wait()
            for k in sorted(desc_c):
                desc_c.pop(k).wait()
            desc_i = (
                pltpu.async_copy(src_hbm.at[wid, pl.ds((ph + 1) * PH, PH)],
                                 srcb[1 - pb], semi),
                pltpu.async_copy(dst_hbm.at[wid, pl.ds((ph + 1) * PH, PH)],
                                 dstb[1 - pb], semi),
            )
        desc_g.pop(c).wait()
        desc_s[c] = pltpu.async_copy(rows[b], sh_sums.at[dstb[pb].at[j]],
                                     sems[b], add=True)
        desc_c[c] = pltpu.async_copy(ones_v, sh_cnt.at[dstb[pb].at[j]],
                                     semc, add=True)
        if c - CNT_LAG in desc_c:
            desc_c.pop(c - CNT_LAG).wait()
        nc = c + LA
        if nc < NCHUNK:
            if nc - NB in desc_s:
                desc_s.pop(nc - NB).wait()
            npb = (nc // PH) & 1
            nj = nc % PH
            if nj == 0 and desc_i is not None:
                desc_i[0].wait()
                desc_i[1].wait()
                desc_i = None
            desc_g[nc] = pltpu.async_copy(x_hbm.at[srcb[npb].at[nj]],
                                          rows[nc % NB], semg[nc % NB])
    for k in sorted(desc_s):
        desc_s.pop(k).wait()
    for k in sorted(desc_c):
        desc_c.pop(k).wait()
    plsc.subcore_barrier()

    for z in range(8):
        w = CH if z < 7 else RPT - 7 * CH
        off = sid * RPT + z * CH
        pltpu.sync_copy(sh_sums.at[pl.ds(off, w)], rows0.at[pl.ds(0, w)])
        pltpu.sync_copy(rows0.at[pl.ds(0, w)], sums_out.at[cid, pl.ds(off, w)])
        pltpu.sync_copy(sh_cnt.at[pl.ds(off, w)], ones_v.at[pl.ds(0, w)])
        pltpu.sync_copy(ones_v.at[pl.ds(0, w)], cnt_out.at[cid, pl.ds(off, w)])


@functools.cache
def _sc_aggregate():
    mesh = plsc.VectorSubcoreMesh(
        core_axis_name="c", subcore_axis_name="s",
        num_cores=NC, num_subcores=NS)
    return pl.kernel(
        _sc_aggregate_impl,
        mesh=mesh,
        out_type=(
            jax.ShapeDtypeStruct((NC, N_PAD, D), jnp.float32),
            jax.ShapeDtypeStruct((NC, N_PAD, 16), jnp.float32),
        ),
        scratch_types=[
            pltpu.VMEM((PH, CH), jnp.int32),
            pltpu.VMEM((PH, CH), jnp.int32),
            pltpu.VMEM((PH, CH), jnp.int32),
            pltpu.VMEM((PH, CH), jnp.int32),
            pltpu.VMEM((CH, D), jnp.float32),
            pltpu.VMEM((CH, D), jnp.float32),
            pltpu.VMEM((CH, D), jnp.float32),
            pltpu.VMEM((CH, 16), jnp.float32),
            pltpu.SemaphoreType.DMA,
            pltpu.SemaphoreType.DMA,
            pltpu.SemaphoreType.DMA,
            pltpu.SemaphoreType.DMA,
            pltpu.SemaphoreType.DMA,
            pltpu.SemaphoreType.DMA,
            pltpu.SemaphoreType.DMA,
            pltpu.SemaphoreType.DMA,
            pltpu.VMEM_SHARED((N_PAD, D), jnp.float32),
            pltpu.VMEM_SHARED((N_PAD, 16), jnp.float32),
        ],
        compiler_params=pltpu.CompilerParams(use_tc_tiling_on_sc=False),
    )


R = 1000


def _tc_body(x_ref, s_ref, c_ref, w2_ref, b2_ref, w3_ref, b3_ref,
             wd1_ref, bd1_ref, wd3_ref, bd3_ref, enc_ref, dec_ref):
    bf = jnp.bfloat16
    f32 = jnp.float32
    sums = s_ref[0] + s_ref[1]
    cnt = c_ref[0, :, 0:1] + c_ref[1, :, 0:1]
    agg = (sums / jnp.maximum(cnt, 1.0)).astype(bf)
    x = x_ref[...].astype(bf)
    e = jnp.maximum(
        jnp.dot(x, w2_ref[0:D, :], preferred_element_type=f32)
        + jnp.dot(agg, w2_ref[D:2 * D, :], preferred_element_type=f32)
        + b2_ref[...], 0.0)
    enc = (jnp.dot(e.astype(bf), w3_ref[...], preferred_element_type=f32)
           + b3_ref[...])
    enc_ref[...] = enc
    dmid = jnp.maximum(
        jnp.dot(enc.astype(bf), wd1_ref[...], preferred_element_type=f32)
        + bd1_ref[...], 0.0)
    dec_ref[...] = (jnp.dot(dmid.astype(bf), wd3_ref[...],
                            preferred_element_type=f32)
                    + bd3_ref[...])


def _tc_forward(interpret=False):
    rep = lambda i: (0, 0)
    rep3 = lambda i: (0, i, 0)
    row = lambda i: (i, 0)
    return pl.pallas_call(
        _tc_body,
        grid=(N_NODES // R,),
        in_specs=[
            pl.BlockSpec((R, D), row),
            pl.BlockSpec((NC, R, D), rep3),
            pl.BlockSpec((NC, R, 16), rep3),
            pl.BlockSpec((2 * D, H2), rep),
            pl.BlockSpec((1, H2), rep),
            pl.BlockSpec((H2, D), rep),
            pl.BlockSpec((1, D), rep),
            pl.BlockSpec((D, H2), rep),
            pl.BlockSpec((1, H2), rep),
            pl.BlockSpec((H2, 2 * D), rep),
            pl.BlockSpec((1, 2 * D), rep),
        ],
        out_specs=[
            pl.BlockSpec((R, D), row),
            pl.BlockSpec((R, 2 * D), row),
        ],
        out_shape=[
            jax.ShapeDtypeStruct((N_NODES, D), jnp.float32),
            jax.ShapeDtypeStruct((N_NODES, 2 * D), jnp.float32),
        ],
        interpret=interpret,
    )


def kernel(x, edge_index, W_enc2, b_enc2, W_enc3, b_enc3,
           W_dec1, b_dec1, W_dec3, b_dec3):
    src = edge_index[0].reshape(NW, NCHUNK, CH)
    dst = edge_index[1].reshape(NW, NCHUNK, CH)
    zrow = jnp.zeros((CH, D), jnp.float32)
    zcnt = jnp.zeros((CH, 16), jnp.float32)
    ones = jnp.ones((CH, 16), jnp.float32)
    sums2, cnt2 = _sc_aggregate()(x, src, dst, zrow, zcnt, ones)
    bf = jnp.bfloat16
    enc, dec = _tc_forward()(
        x, sums2, cnt2,
        W_enc2.astype(bf), b_enc2.reshape(1, H2),
        W_enc3.astype(bf), b_enc3.reshape(1, D),
        W_dec1.astype(bf), b_dec1.reshape(1, H2),
        W_dec3.astype(bf), b_dec3.reshape(1, 2 * D),
    )
    return enc, dec


# speedup vs baseline: 1.0188x; 1.0188x over previous
import functools

import jax
import jax.numpy as jnp
from jax import lax
from jax.experimental import pallas as pl
from jax.experimental.pallas import tpu as pltpu
from jax.experimental.pallas import tpu_sc as plsc

N_NODES = 10000
N_EDGES = 320000
D = 128
H2 = 768
NC, NS = 2, 16
NW = NC * NS
EPW = N_EDGES // NW
CH = 80
NCHUNK = EPW // CH
N_PAD = 10112
RPT = N_PAD // NS
NB = 3
LA = NB - 1

PH = 25
NPH = NCHUNK // PH
CNT_LAG = 6


def _sc_aggregate_impl(x_hbm, src_hbm, dst_hbm, zrow_hbm, zcnt_hbm, ones_hbm,
                       sums_out, cnt_out,
                       srcb0, srcb1, dstb0, dstb1, rows0, rows1, rows2, ones_v,
                       semg0, semg1, semg2, sems0, sems1, sems2, semi, semc,
                       sh_sums, sh_cnt):
    cid = lax.axis_index("c")
    sid = lax.axis_index("s")
    wid = sid * NC + cid
    srcb = (srcb0, srcb1)
    dstb = (dstb0, dstb1)
    rows = (rows0, rows1, rows2)
    semg = (semg0, semg1, semg2)
    sems = (sems0, sems1, sems2)

    pltpu.sync_copy(zrow_hbm, rows0)
    pltpu.sync_copy(zcnt_hbm, ones_v)
    zdesc = []
    for z in range(8):
        w = CH if z < 7 else RPT - 7 * CH
        off = sid * RPT + z * CH
        zdesc.append(pltpu.async_copy(rows0.at[pl.ds(0, w)],
                                      sh_sums.at[pl.ds(off, w)], semg0))
        zdesc.append(pltpu.async_copy(ones_v.at[pl.ds(0, w)],
                                      sh_cnt.at[pl.ds(off, w)], semc))
    for d in zdesc:
        d.wait()
    pltpu.sync_copy(ones_hbm, ones_v)
    plsc.subcore_barrier()

    pltpu.sync_copy(src_hbm.at[wid, pl.ds(0, PH)], srcb0)
    pltpu.sync_copy(dst_hbm.at[wid, pl.ds(0, PH)], dstb0)
    desc_g = {}
    desc_s = {}
    desc_c = {}
    desc_i = None
    for k in range(LA):
        desc_g[k] = pltpu.async_copy(x_hbm.at[srcb0.at[k]],
                                     rows[k % NB], semg[k % NB])
    for c in range(NCHUNK):
        b = c % NB
        ph = c // PH
        pb = ph & 1
        j = c % PH
        if j == 0 and ph + 1 < NPH:
            for k in sorted(desc_s):
                desc_s.pop(k).wait()
            for k in sorted(desc_c):
                desc_c.pop(k).wait()
            desc_i = (
                pltpu.async_copy(src_hbm.at[wid, pl.ds((ph + 1) * PH, PH)],
                                 srcb[1 - pb], semi),
                pltpu.async_copy(dst_hbm.at[wid, pl.ds((ph + 1) * PH, PH)],
                                 dstb[1 - pb], semi),
            )
        desc_g.pop(c).wait()
        desc_s[c] = pltpu.async_copy(rows[b], sh_sums.at[dstb[pb].at[j]],
                                     sems[b], add=True)
        desc_c[c] = pltpu.async_copy(ones_v, sh_cnt.at[dstb[pb].at[j]],
                                     semc, add=True)
        if c - CNT_LAG in desc_c:
            desc_c.pop(c - CNT_LAG).wait()
        nc = c + LA
        if nc < NCHUNK:
            if nc - NB in desc_s:
                desc_s.pop(nc - NB).wait()
            npb = (nc // PH) & 1
            nj = nc % PH
            if nj == 0 and desc_i is not None:
                desc_i[0].wait()
                desc_i[1].wait()
                desc_i = None
            desc_g[nc] = pltpu.async_copy(x_hbm.at[srcb[npb].at[nj]],
                                          rows[nc % NB], semg[nc % NB])
    for k in sorted(desc_s):
        desc_s.pop(k).wait()
    for k in sorted(desc_c):
        desc_c.pop(k).wait()
    plsc.subcore_barrier()

    wdesc = {}
    for z in range(8):
        w = CH if z < 7 else RPT - 7 * CH
        off = sid * RPT + z * CH
        rb = rows[z % NB]
        if z - NB in wdesc:
            wdesc.pop(z - NB).wait()
        pltpu.async_copy(sh_sums.at[pl.ds(off, w)],
                         rb.at[pl.ds(0, w)], semg[z % NB]).wait()
        wdesc[z] = pltpu.async_copy(rb.at[pl.ds(0, w)],
                                    sums_out.at[cid, pl.ds(off, w)],
                                    sems[z % NB])
        pltpu.sync_copy(sh_cnt.at[pl.ds(off, w)], ones_v.at[pl.ds(0, w)])
        pltpu.sync_copy(ones_v.at[pl.ds(0, w)], cnt_out.at[cid, pl.ds(off, w)])
    for k in sorted(wdesc):
        wdesc.pop(k).wait()


@functools.cache
def _sc_aggregate():
    mesh = plsc.VectorSubcoreMesh(
        core_axis_name="c", subcore_axis_name="s",
        num_cores=NC, num_subcores=NS)
    return pl.kernel(
        _sc_aggregate_impl,
        mesh=mesh,
        out_type=(
            jax.ShapeDtypeStruct((NC, N_PAD, D), jnp.float32),
            jax.ShapeDtypeStruct((NC, N_PAD, 16), jnp.float32),
        ),
        scratch_types=[
            pltpu.VMEM((PH, CH), jnp.int32),
            pltpu.VMEM((PH, CH), jnp.int32),
            pltpu.VMEM((PH, CH), jnp.int32),
            pltpu.VMEM((PH, CH), jnp.int32),
            pltpu.VMEM((CH, D), jnp.float32),
            pltpu.VMEM((CH, D), jnp.float32),
            pltpu.VMEM((CH, D), jnp.float32),
            pltpu.VMEM((CH, 16), jnp.float32),
            pltpu.SemaphoreType.DMA,
            pltpu.SemaphoreType.DMA,
            pltpu.SemaphoreType.DMA,
            pltpu.SemaphoreType.DMA,
            pltpu.SemaphoreType.DMA,
            pltpu.SemaphoreType.DMA,
            pltpu.SemaphoreType.DMA,
            pltpu.SemaphoreType.DMA,
            pltpu.VMEM_SHARED((N_PAD, D), jnp.float32),
            pltpu.VMEM_SHARED((N_PAD, 16), jnp.float32),
        ],
        compiler_params=pltpu.CompilerParams(use_tc_tiling_on_sc=False),
    )


R = 1000


def _tc_body(x_ref, s_ref, c_ref, w2_ref, b2_ref, w3_ref, b3_ref,
             wd1_ref, bd1_ref, wd3_ref, bd3_ref, enc_ref, dec_ref):
    bf = jnp.bfloat16
    f32 = jnp.float32
    sums = s_ref[0] + s_ref[1]
    cnt = c_ref[0, :, 0:1] + c_ref[1, :, 0:1]
    agg = (sums / jnp.maximum(cnt, 1.0)).astype(bf)
    x = x_ref[...].astype(bf)
    e = jnp.maximum(
        jnp.dot(x, w2_ref[0:D, :], preferred_element_type=f32)
        + jnp.dot(agg, w2_ref[D:2 * D, :], preferred_element_type=f32)
        + b2_ref[...], 0.0)
    enc = (jnp.dot(e.astype(bf), w3_ref[...], preferred_element_type=f32)
           + b3_ref[...])
    enc_ref[...] = enc
    dmid = jnp.maximum(
        jnp.dot(enc.astype(bf), wd1_ref[...], preferred_element_type=f32)
        + bd1_ref[...], 0.0)
    dec_ref[...] = (jnp.dot(dmid.astype(bf), wd3_ref[...],
                            preferred_element_type=f32)
                    + bd3_ref[...])


def _tc_forward(interpret=False):
    rep = lambda i: (0, 0)
    rep3 = lambda i: (0, i, 0)
    row = lambda i: (i, 0)
    return pl.pallas_call(
        _tc_body,
        grid=(N_NODES // R,),
        in_specs=[
            pl.BlockSpec((R, D), row),
            pl.BlockSpec((NC, R, D), rep3),
            pl.BlockSpec((NC, R, 16), rep3),
            pl.BlockSpec((2 * D, H2), rep),
            pl.BlockSpec((1, H2), rep),
            pl.BlockSpec((H2, D), rep),
            pl.BlockSpec((1, D), rep),
            pl.BlockSpec((D, H2), rep),
            pl.BlockSpec((1, H2), rep),
            pl.BlockSpec((H2, 2 * D), rep),
            pl.BlockSpec((1, 2 * D), rep),
        ],
        out_specs=[
            pl.BlockSpec((R, D), row),
            pl.BlockSpec((R, 2 * D), row),
        ],
        out_shape=[
            jax.ShapeDtypeStruct((N_NODES, D), jnp.float32),
            jax.ShapeDtypeStruct((N_NODES, 2 * D), jnp.float32),
        ],
        interpret=interpret,
    )


def kernel(x, edge_index, W_enc2, b_enc2, W_enc3, b_enc3,
           W_dec1, b_dec1, W_dec3, b_dec3):
    src = edge_index[0].reshape(NW, NCHUNK, CH)
    dst = edge_index[1].reshape(NW, NCHUNK, CH)
    zrow = jnp.zeros((CH, D), jnp.float32)
    zcnt = jnp.zeros((CH, 16), jnp.float32)
    ones = jnp.ones((CH, 16), jnp.float32)
    sums2, cnt2 = _sc_aggregate()(x, src, dst, zrow, zcnt, ones)
    bf = jnp.bfloat16
    enc, dec = _tc_forward()(
        x, sums2, cnt2,
        W_enc2.astype(bf), b_enc2.reshape(1, H2),
        W_enc3.astype(bf), b_enc3.reshape(1, D),
        W_dec1.astype(bf), b_dec1.reshape(1, H2),
        W_dec3.astype(bf), b_dec3.reshape(1, 2 * D),
    )
    return enc, dec
